# Initial kernel scaffold; baseline (speedup 1.0000x reference)
import jax
import jax.numpy as jnp
from jax.experimental import pallas as pl


def kernel(emb_w, tag_w, per_w, grp_w, lin_w, lin_b, lin_e_w, lin_e_b, period_table, group_table, z, tag, rel_pos, edge_attr):
    raise NotImplementedError("write your pallas kernel here")


# speedup vs baseline: 4.1609x; 4.1609x over previous
import jax
import jax.numpy as jnp
from jax import lax
from jax.experimental import pallas as pl
from jax.experimental.pallas import tpu as pltpu

FUSED_VOCAB = 128
MAX_TILE = 8192


def _round_up(x, m):
    return ((x + m - 1) // m) * m


def _plan(n, e):
    g = max(1, -(-e // MAX_TILE), -(-n // MAX_TILE))
    tn = _round_up(max(1, -(-n // g)), 8)
    te = _round_up(max(1, -(-e // g)), 8)
    return g, tn, te


def _fused_kernel(tag_off_ref, z_ref, t_ref, c_ref, rp_ref, ea_ref,
                  w3_ref, w16_ref, be_ref, h_ref, e_ref):
    tile_n = z_ref.shape[0]
    lanes = lax.broadcasted_iota(jnp.int32, (tile_n, FUSED_VOCAB), 1)
    mh = (lanes == z_ref[...]) | (lanes == t_ref[...] + tag_off_ref[0])
    h_ref[...] = jnp.dot(mh.astype(jnp.float32), c_ref[...],
                         preferred_element_type=jnp.float32)
    e_ref[...] = (jnp.dot(rp_ref[...], w3_ref[...],
                          preferred_element_type=jnp.float32)
                  + jnp.dot(ea_ref[...], w16_ref[...],
                            preferred_element_type=jnp.float32)
                  + be_ref[...])


def kernel(emb_w, tag_w, per_w, grp_w, lin_w, lin_b, lin_e_w, lin_e_b,
           period_table, group_table, z, tag, rel_pos, edge_attr):
    n = z.shape[0]
    e = rel_pos.shape[0]
    n_elements = emb_w.shape[0]
    n_tags = tag_w.shape[0]
    atom_dim = emb_w.shape[1]
    tag_dim = tag_w.shape[1]
    pg_dim = per_w.shape[1]
    hidden = lin_w.shape[1]
    rp_dim = rel_pos.shape[1]

    emb_eff = jnp.dot(emb_w, lin_w[:atom_dim], preferred_element_type=jnp.float32)
    tag_eff = jnp.dot(tag_w, lin_w[atom_dim:atom_dim + tag_dim],
                      preferred_element_type=jnp.float32)
    per_eff = jnp.dot(per_w, lin_w[atom_dim + tag_dim:atom_dim + tag_dim + pg_dim],
                      preferred_element_type=jnp.float32)
    grp_eff = jnp.dot(grp_w, lin_w[atom_dim + tag_dim + pg_dim:],
                      preferred_element_type=jnp.float32)
    a_rows = (emb_eff + per_eff[period_table] + grp_eff[group_table]
              + lin_b.astype(jnp.float32))
    c = jnp.zeros((FUSED_VOCAB, hidden), jnp.float32)
    c = lax.dynamic_update_slice(c, a_rows, (0, 0))
    c = lax.dynamic_update_slice(c, tag_eff, (n_elements, 0))
    tag_off = jnp.full((1,), n_elements, jnp.int32)

    w3 = lin_e_w[:rp_dim].astype(jnp.float32)
    w16 = lin_e_w[rp_dim:].astype(jnp.float32)

    g, tn, te = _plan(n, e)
    n_pad, e_pad = g * tn, g * te
    zc = z.astype(jnp.int32)
    tc = tag.astype(jnp.int32)
    rp = rel_pos.astype(jnp.float32)
    ea = edge_attr.astype(jnp.float32)
    if n_pad != n:
        zc = jnp.pad(zc, (0, n_pad - n))
        tc = jnp.pad(tc, (0, n_pad - n))
    if e_pad != e:
        rp = jnp.pad(rp, ((0, e_pad - e), (0, 0)))
        ea = jnp.pad(ea, ((0, e_pad - e), (0, 0)))
    zc = zc.reshape(n_pad, 1)
    tc = tc.reshape(n_pad, 1)

    h_full, e_full = pl.pallas_call(
        _fused_kernel,
        out_shape=(jax.ShapeDtypeStruct((n_pad, hidden), jnp.float32),
                   jax.ShapeDtypeStruct((e_pad, hidden), jnp.float32)),
        grid=(g,),
        in_specs=[
            pl.BlockSpec(memory_space=pltpu.SMEM),
            pl.BlockSpec((tn, 1), lambda i: (i, 0)),
            pl.BlockSpec((tn, 1), lambda i: (i, 0)),
            pl.BlockSpec((FUSED_VOCAB, hidden), lambda i: (0, 0)),
            pl.BlockSpec((te, rp_dim), lambda i: (i, 0)),
            pl.BlockSpec((te, ea.shape[1]), lambda i: (i, 0)),
            pl.BlockSpec((rp_dim, hidden), lambda i: (0, 0)),
            pl.BlockSpec((ea.shape[1], hidden), lambda i: (0, 0)),
            pl.BlockSpec((1, hidden), lambda i: (0, 0)),
        ],
        out_specs=(pl.BlockSpec((tn, hidden), lambda i: (i, 0)),
                   pl.BlockSpec((te, hidden), lambda i: (i, 0))),
        compiler_params=pltpu.CompilerParams(
            dimension_semantics=("parallel",)),
    )(tag_off, zc, tc, c, rp, ea, w3, w16, lin_e_b.astype(jnp.float32))

    h = h_full if n_pad == n else h_full[:n]
    e_out = e_full if e_pad == e else e_full[:e]
    return h, e_out
